# Initial kernel scaffold; baseline (speedup 1.0000x reference)
import functools

import jax
import jax.numpy as jnp
from jax import lax
from jax.experimental import pallas as pl
from jax.experimental.pallas import tpu as pltpu
from jax.experimental.pallas import tpu_sc as plsc

N = 50000
E = 800000
D = 64
H = 32
NPAD = 50176
EPAD = 819200
ROWS = EPAD // 128
TROWS = ROWS // 16
CHUNK_ROWS = 8
NCHUNK = TROWS // CHUNK_ROWS
NODE_T = NPAD // 16


def _sc_aggregate(h_cat, srcp_cat, srcn_cat, dstp2, dstn2, ones_h, zeros_h,
                  zerosd_h):
    mesh = plsc.VectorSubcoreMesh(core_axis_name="c", subcore_axis_name="s")

    @functools.partial(
        pl.kernel,
        out_type=(
            jax.ShapeDtypeStruct((2, NPAD, H), jnp.float32),
            jax.ShapeDtypeStruct((2, NPAD, H), jnp.float32),
            jax.ShapeDtypeStruct((NPAD, 1), jnp.float32),
            jax.ShapeDtypeStruct((NPAD, 1), jnp.float32),
        ),
        mesh=mesh,
        scratch_types=(
            pltpu.VMEM_SHARED((NPAD, H), jnp.float32),
            pltpu.VMEM_SHARED((NPAD, 1), jnp.float32),
            pltpu.VMEM((CHUNK_ROWS * 128,), jnp.int32),
            pltpu.VMEM((CHUNK_ROWS, 128), jnp.int32),
            pltpu.VMEM((CHUNK_ROWS, 128, H), jnp.float32),
            pltpu.VMEM((128, 1), jnp.float32),
            pltpu.SemaphoreType.DMA,
        ),
    )
    def sc_kernel(h_cat, srcp_cat, srcn_cat, dstp2, dstn2, ones_h, zeros_h,
                  zerosd_h, agg_p, agg_n, deg_p, deg_n,
                  acc, dega, srcv, dstv, gbuf, onesv, sem):
        cid = lax.axis_index("c")
        sid = lax.axis_index("s")
        nodebase = sid * NODE_T

        pltpu.sync_copy(ones_h, onesv)
        pltpu.sync_copy(zeros_h, acc.at[pl.ds(nodebase, NODE_T)])
        pltpu.sync_copy(zerosd_h, dega.at[pl.ds(nodebase, NODE_T)])
        plsc.subcore_barrier()

        def run_phase(src_cat, dst2, deg_core):
            def body(i, carry):
                eoff = cid * EPAD + sid * (TROWS * 128) + i * (CHUNK_ROWS * 128)
                roff = sid * TROWS + i * CHUNK_ROWS
                pltpu.sync_copy(src_cat.at[pl.ds(eoff, CHUNK_ROWS * 128)], srcv)
                pltpu.sync_copy(dst2.at[pl.ds(roff, CHUNK_ROWS)], dstv)
                cps = [
                    pltpu.async_copy(
                        h_cat.at[srcv.at[pl.ds(j * 128, 128)]],
                        gbuf.at[j], sem)
                    for j in range(CHUNK_ROWS)
                ]
                for c in cps:
                    c.wait()
                for j in range(CHUNK_ROWS):
                    pltpu.sync_copy(gbuf.at[j], acc.at[dstv.at[j]], add=True)

                @pl.when(cid == deg_core)
                def _():
                    for j in range(CHUNK_ROWS):
                        pltpu.sync_copy(onesv, dega.at[dstv.at[j]], add=True)

                return carry

            lax.fori_loop(0, NCHUNK, body, 0)

        run_phase(srcp_cat, dstp2, 0)
        plsc.subcore_barrier()
        pltpu.sync_copy(acc.at[pl.ds(nodebase, NODE_T)],
                        agg_p.at[cid, pl.ds(nodebase, NODE_T)])

        @pl.when(cid == 0)
        def _():
            pltpu.sync_copy(dega.at[pl.ds(nodebase, NODE_T)],
                            deg_p.at[pl.ds(nodebase, NODE_T)])

        pltpu.sync_copy(zeros_h, acc.at[pl.ds(nodebase, NODE_T)])
        plsc.subcore_barrier()

        run_phase(srcn_cat, dstn2, 1)
        plsc.subcore_barrier()
        pltpu.sync_copy(acc.at[pl.ds(nodebase, NODE_T)],
                        agg_n.at[cid, pl.ds(nodebase, NODE_T)])

        @pl.when(cid == 1)
        def _():
            pltpu.sync_copy(dega.at[pl.ds(nodebase, NODE_T)],
                            deg_n.at[pl.ds(nodebase, NODE_T)])

    return sc_kernel(h_cat, srcp_cat, srcn_cat, dstp2, dstn2, ones_h, zeros_h,
                     zerosd_h)


BLK = 256


def _tc_body(aggp_ref, aggn_ref, degp_ref, degn_ref, wp_ref, bp_ref, wn_ref,
             bn_ref, wpp_ref, bpp_ref, wpn_ref, bpn_ref, out_ref):
    dp = jnp.maximum(degp_ref[...], 1.0)
    dn = jnp.maximum(degn_ref[...], 1.0)
    ap0 = aggp_ref[0] / dp
    ap1 = aggp_ref[1] / dp
    an0 = aggn_ref[0] / dn
    an1 = aggn_ref[1] / dn
    wp = wp_ref[...]
    wn = wn_ref[...]
    f32 = jnp.float32
    hp = (jnp.dot(ap0, wp[:H, :], preferred_element_type=f32)
          + jnp.dot(ap1, wp[H:, :], preferred_element_type=f32) + bp_ref[...])
    hn = (jnp.dot(an0, wn[:H, :], preferred_element_type=f32)
          + jnp.dot(an1, wn[H:, :], preferred_element_type=f32) + bn_ref[...])
    out_ref[...] = (jnp.dot(hp, wpp_ref[...], preferred_element_type=f32)
                    + bpp_ref[...]
                    + jnp.dot(hn, wpn_ref[...], preferred_element_type=f32)
                    + bpn_ref[...])


def _tc_combine(agg_p, agg_n, deg_p, deg_n, W_pos, b_pos, W_neg, b_neg,
                W_psi_pos, b_psi_pos, W_psi_neg, b_psi_neg):
    grid = (NPAD // BLK,)
    full = lambda shape: pl.BlockSpec(shape, lambda i: (0,) * len(shape))
    return pl.pallas_call(
        _tc_body,
        grid=grid,
        in_specs=[
            pl.BlockSpec((2, BLK, H), lambda i: (0, i, 0)),
            pl.BlockSpec((2, BLK, H), lambda i: (0, i, 0)),
            pl.BlockSpec((BLK, 1), lambda i: (i, 0)),
            pl.BlockSpec((BLK, 1), lambda i: (i, 0)),
            full((D, D)), full((1, D)),
            full((D, D)), full((1, D)),
            full((D, D)), full((1, D)),
            full((D, D)), full((1, D)),
        ],
        out_specs=pl.BlockSpec((BLK, D), lambda i: (i, 0)),
        out_shape=jax.ShapeDtypeStruct((NPAD, D), jnp.float32),
    )(agg_p, agg_n, deg_p, deg_n, W_pos, b_pos.reshape(1, D), W_neg,
      b_neg.reshape(1, D), W_psi_pos, b_psi_pos.reshape(1, D), W_psi_neg,
      b_psi_neg.reshape(1, D))


def _prep_edges(edge_index):
    src = edge_index[0].astype(jnp.int32)
    dst = edge_index[1].astype(jnp.int32)
    srcp = jnp.zeros((EPAD,), jnp.int32).at[:E].set(src)
    dstp = jnp.full((EPAD,), N, jnp.int32).at[:E].set(dst)
    src_cat = jnp.concatenate([srcp, srcp + NPAD])
    dst2 = dstp.reshape(ROWS, 128)
    return src_cat, dst2


def kernel(t, h, edge_index_pos, edge_index_neg, W_pos, b_pos, W_neg, b_neg,
           W_psi_pos, b_psi_pos, W_psi_neg, b_psi_neg):
    h_cat = jnp.zeros((2 * NPAD, H), jnp.float32)
    h_cat = h_cat.at[:N, :].set(h[:, :H]).at[NPAD:NPAD + N, :].set(h[:, H:])
    srcp_cat, dstp2 = _prep_edges(edge_index_pos)
    srcn_cat, dstn2 = _prep_edges(edge_index_neg)
    ones_h = jnp.ones((128, 1), jnp.float32)
    zeros_h = jnp.zeros((NODE_T, H), jnp.float32)
    zerosd_h = jnp.zeros((NODE_T, 1), jnp.float32)

    agg_p, agg_n, deg_p, deg_n = _sc_aggregate(
        h_cat, srcp_cat, srcn_cat, dstp2, dstn2, ones_h, zeros_h, zerosd_h)

    out = _tc_combine(agg_p, agg_n, deg_p, deg_n, W_pos, b_pos, W_neg, b_neg,
                      W_psi_pos, b_psi_pos, W_psi_neg, b_psi_neg)
    return out[:N]


# speedup vs baseline: 5.9327x; 5.9327x over previous
import functools

import jax
import jax.numpy as jnp
from jax import lax
from jax.experimental import pallas as pl
from jax.experimental.pallas import tpu as pltpu
from jax.experimental.pallas import tpu_sc as plsc

N = 50000
E = 800000
D = 64
H = 32
DW = 8
NPAD = 50176
EPAD = 819200
TEDGE = EPAD // 16
L = 400
NCH = TEDGE // L
NBODY = NCH // 2
NODE_T = NPAD // 16


def _sc_aggregate(h_cat, srcp_cat, srcn_cat, dstp, dstn, ones_h, zeros_h):
    mesh = plsc.VectorSubcoreMesh(core_axis_name="c", subcore_axis_name="s")

    @functools.partial(
        pl.kernel,
        out_type=(
            jax.ShapeDtypeStruct((2, NPAD, H), jnp.float32),
            jax.ShapeDtypeStruct((2, NPAD, H), jnp.float32),
            jax.ShapeDtypeStruct((NPAD, DW), jnp.float32),
            jax.ShapeDtypeStruct((NPAD, DW), jnp.float32),
        ),
        mesh=mesh,
        compiler_params=pltpu.CompilerParams(use_tc_tiling_on_sc=False),
        scratch_types=(
            pltpu.VMEM_SHARED((NPAD, H), jnp.float32),
            pltpu.VMEM((4, L), jnp.int32),
            pltpu.VMEM((4, L), jnp.int32),
            pltpu.VMEM((L, H), jnp.float32),
            pltpu.VMEM((L, H), jnp.float32),
            pltpu.SemaphoreType.DMA,
            pltpu.SemaphoreType.DMA,
            pltpu.SemaphoreType.DMA,
            pltpu.SemaphoreType.DMA,
            pltpu.SemaphoreType.DMA,
            pltpu.SemaphoreType.DMA,
        ),
    )
    def sc_kernel(h_cat, srcp_cat, srcn_cat, dstp, dstn, ones_h, zeros_h,
                  agg_p, agg_n, deg_p, deg_n,
                  acc, srcbig, dstbig, gbuf0, gbuf1,
                  gsem0, gsem1, ssem0, ssem1, isem0, isem1):
        cid = lax.axis_index("c")
        sid = lax.axis_index("s")
        nodebase = sid * NODE_T

        pltpu.sync_copy(zeros_h, acc.at[pl.ds(nodebase, NODE_T)])
        plsc.subcore_barrier()

        def run_phase(src_cat, dst1):
            ebase = cid * EPAD + sid * TEDGE
            dbase = sid * TEDGE

            def iload(i, isem, sync=False):
                slot = lax.rem(i, 4)
                if sync:
                    pltpu.sync_copy(src_cat.at[pl.ds(ebase + i * L, L)],
                                    srcbig.at[slot])
                    pltpu.sync_copy(dst1.at[pl.ds(dbase + i * L, L)],
                                    dstbig.at[slot])
                else:
                    pltpu.async_copy(src_cat.at[pl.ds(ebase + i * L, L)],
                                     srcbig.at[slot], isem)
                    pltpu.async_copy(dst1.at[pl.ds(dbase + i * L, L)],
                                     dstbig.at[slot], isem)

            def drain_i(i, isem):
                slot = lax.rem(i, 4)
                pltpu.make_async_copy(src_cat.at[pl.ds(ebase + i * L, L)],
                                      srcbig.at[slot], isem).wait()
                pltpu.make_async_copy(dst1.at[pl.ds(dbase + i * L, L)],
                                      dstbig.at[slot], isem).wait()

            def fire_g(i, gb, sem):
                pltpu.async_copy(h_cat.at[srcbig.at[lax.rem(i, 4)]], gb, sem)

            def drain_g(i, gb, sem):
                pltpu.make_async_copy(h_cat.at[srcbig.at[lax.rem(i, 4)]],
                                      gb, sem).wait()

            def fire_s(i, gb, sem):
                pltpu.async_copy(gb, acc.at[dstbig.at[lax.rem(i, 4)]], sem,
                                 add=True)

            def drain_s(i, gb, sem):
                pltpu.make_async_copy(gb, acc.at[dstbig.at[lax.rem(i, 4)]],
                                      sem).wait()

            iload(0, None, sync=True)
            iload(1, None, sync=True)
            iload(2, isem0)
            iload(3, isem1)
            fire_g(0, gbuf0, gsem0)
            fire_g(1, gbuf1, gsem1)

            def body(k2, carry):
                i0 = 2 * k2
                drain_g(i0, gbuf0, gsem0)
                fire_s(i0, gbuf0, ssem0)
                drain_g(i0 + 1, gbuf1, gsem1)
                fire_s(i0 + 1, gbuf1, ssem1)

                @pl.when(k2 < NBODY - 1)
                def _():
                    drain_s(i0, gbuf0, ssem0)
                    drain_i(i0 + 2, isem0)
                    fire_g(i0 + 2, gbuf0, gsem0)

                    @pl.when(k2 < NBODY - 2)
                    def _():
                        iload(i0 + 4, isem0)

                    drain_s(i0 + 1, gbuf1, ssem1)
                    drain_i(i0 + 3, isem1)
                    fire_g(i0 + 3, gbuf1, gsem1)

                    @pl.when(k2 < NBODY - 2)
                    def _():
                        iload(i0 + 5, isem1)

                return carry

            lax.fori_loop(0, NBODY, body, 0)
            drain_s(NCH - 2, gbuf0, ssem0)
            drain_s(NCH - 1, gbuf1, ssem1)

        run_phase(srcp_cat, dstp)
        plsc.subcore_barrier()
        pltpu.sync_copy(acc.at[pl.ds(nodebase, NODE_T)],
                        agg_p.at[cid, pl.ds(nodebase, NODE_T)])
        pltpu.sync_copy(zeros_h, acc.at[pl.ds(nodebase, NODE_T)])
        plsc.subcore_barrier()

        run_phase(srcn_cat, dstn)
        plsc.subcore_barrier()
        pltpu.sync_copy(acc.at[pl.ds(nodebase, NODE_T)],
                        agg_n.at[cid, pl.ds(nodebase, NODE_T)])
        pltpu.sync_copy(zeros_h, acc.at[pl.ds(nodebase, NODE_T)])
        plsc.subcore_barrier()

        def deg_phase(dst1):
            dbase = sid * TEDGE

            def iload_d(i, isem, sync=False):
                slot = lax.rem(i, 4)
                if sync:
                    pltpu.sync_copy(dst1.at[pl.ds(dbase + i * L, L)],
                                    dstbig.at[slot])
                else:
                    pltpu.async_copy(dst1.at[pl.ds(dbase + i * L, L)],
                                     dstbig.at[slot], isem)

            def drain_i(i, isem):
                pltpu.make_async_copy(dst1.at[pl.ds(dbase + i * L, L)],
                                      dstbig.at[lax.rem(i, 4)], isem).wait()

            def fire_s(i, gb, sem):
                pltpu.async_copy(gb, acc.at[dstbig.at[lax.rem(i, 4)]], sem,
                                 add=True)

            def drain_s(i, gb, sem):
                pltpu.make_async_copy(gb, acc.at[dstbig.at[lax.rem(i, 4)]],
                                      sem).wait()

            pltpu.sync_copy(ones_h, gbuf0)
            pltpu.sync_copy(ones_h, gbuf1)
            iload_d(0, None, sync=True)
            iload_d(1, None, sync=True)
            iload_d(2, isem0)
            iload_d(3, isem1)

            def body(k2, carry):
                i0 = 2 * k2
                fire_s(i0, gbuf0, ssem0)
                fire_s(i0 + 1, gbuf1, ssem1)

                @pl.when(k2 < NBODY - 1)
                def _():
                    drain_s(i0, gbuf0, ssem0)
                    drain_i(i0 + 2, isem0)

                    @pl.when(k2 < NBODY - 2)
                    def _():
                        iload_d(i0 + 4, isem0)

                    drain_s(i0 + 1, gbuf1, ssem1)
                    drain_i(i0 + 3, isem1)

                    @pl.when(k2 < NBODY - 2)
                    def _():
                        iload_d(i0 + 5, isem1)

                return carry

            lax.fori_loop(0, NBODY, body, 0)
            drain_s(NCH - 2, gbuf0, ssem0)
            drain_s(NCH - 1, gbuf1, ssem1)

        @pl.when(cid == 0)
        def _():
            deg_phase(dstp)

        @pl.when(cid == 1)
        def _():
            deg_phase(dstn)

        plsc.subcore_barrier()

        @pl.when(cid == 0)
        def _():
            pltpu.sync_copy(acc.at[pl.ds(nodebase, NODE_T), pl.ds(0, DW)],
                            deg_p.at[pl.ds(nodebase, NODE_T)])

        @pl.when(cid == 1)
        def _():
            pltpu.sync_copy(acc.at[pl.ds(nodebase, NODE_T), pl.ds(0, DW)],
                            deg_n.at[pl.ds(nodebase, NODE_T)])

    return sc_kernel(h_cat, srcp_cat, srcn_cat, dstp, dstn, ones_h, zeros_h)


BLK = 256


def _tc_body(aggp_ref, aggn_ref, degp_ref, degn_ref, wp_ref, bp_ref, wn_ref,
             bn_ref, wpp_ref, bpp_ref, wpn_ref, bpn_ref, out_ref):
    dp = jnp.maximum(degp_ref[...][:, :1], 1.0)
    dn = jnp.maximum(degn_ref[...][:, :1], 1.0)
    ap0 = aggp_ref[0] / dp
    ap1 = aggp_ref[1] / dp
    an0 = aggn_ref[0] / dn
    an1 = aggn_ref[1] / dn
    wp = wp_ref[...]
    wn = wn_ref[...]
    f32 = jnp.float32
    hp = (jnp.dot(ap0, wp[:H, :], preferred_element_type=f32)
          + jnp.dot(ap1, wp[H:, :], preferred_element_type=f32) + bp_ref[...])
    hn = (jnp.dot(an0, wn[:H, :], preferred_element_type=f32)
          + jnp.dot(an1, wn[H:, :], preferred_element_type=f32) + bn_ref[...])
    out_ref[...] = (jnp.dot(hp, wpp_ref[...], preferred_element_type=f32)
                    + bpp_ref[...]
                    + jnp.dot(hn, wpn_ref[...], preferred_element_type=f32)
                    + bpn_ref[...])


def _tc_combine(agg_p, agg_n, deg_p, deg_n, W_pos, b_pos, W_neg, b_neg,
                W_psi_pos, b_psi_pos, W_psi_neg, b_psi_neg):
    grid = (NPAD // BLK,)
    full = lambda shape: pl.BlockSpec(shape, lambda i: (0,) * len(shape))
    return pl.pallas_call(
        _tc_body,
        grid=grid,
        in_specs=[
            pl.BlockSpec((2, BLK, H), lambda i: (0, i, 0)),
            pl.BlockSpec((2, BLK, H), lambda i: (0, i, 0)),
            pl.BlockSpec((BLK, DW), lambda i: (i, 0)),
            pl.BlockSpec((BLK, DW), lambda i: (i, 0)),
            full((D, D)), full((1, D)),
            full((D, D)), full((1, D)),
            full((D, D)), full((1, D)),
            full((D, D)), full((1, D)),
        ],
        out_specs=pl.BlockSpec((BLK, D), lambda i: (i, 0)),
        out_shape=jax.ShapeDtypeStruct((NPAD, D), jnp.float32),
    )(agg_p, agg_n, deg_p, deg_n, W_pos, b_pos.reshape(1, D), W_neg,
      b_neg.reshape(1, D), W_psi_pos, b_psi_pos.reshape(1, D), W_psi_neg,
      b_psi_neg.reshape(1, D))


def _prep_edges(edge_index):
    src = edge_index[0].astype(jnp.int32)
    dst = edge_index[1].astype(jnp.int32)
    srcp = jnp.zeros((EPAD,), jnp.int32).at[:E].set(src)
    dstp = jnp.full((EPAD,), N, jnp.int32).at[:E].set(dst)
    src_cat = jnp.concatenate([srcp, srcp + NPAD])
    return src_cat, dstp


def kernel(t, h, edge_index_pos, edge_index_neg, W_pos, b_pos, W_neg, b_neg,
           W_psi_pos, b_psi_pos, W_psi_neg, b_psi_neg):
    h_cat = jnp.zeros((2 * NPAD, H), jnp.float32)
    h_cat = h_cat.at[:N, :].set(h[:, :H]).at[NPAD:NPAD + N, :].set(h[:, H:])
    srcp_cat, dstp = _prep_edges(edge_index_pos)
    srcn_cat, dstn = _prep_edges(edge_index_neg)
    ones_h = jnp.ones((L, H), jnp.float32)
    zeros_h = jnp.zeros((NODE_T, H), jnp.float32)

    agg_p, agg_n, deg_p, deg_n = _sc_aggregate(
        h_cat, srcp_cat, srcn_cat, dstp, dstn, ones_h, zeros_h)

    out = _tc_combine(agg_p, agg_n, deg_p, deg_n, W_pos, b_pos, W_neg, b_neg,
                      W_psi_pos, b_psi_pos, W_psi_neg, b_psi_neg)
    return out[:N]
